# baseline (speedup 1.0000x reference)
import functools

import jax
import jax.numpy as jnp
from jax import lax
from jax.experimental import pallas as pl
from jax.experimental.pallas import tpu as pltpu
from jax.experimental.pallas import tpu_sc as plsc

VOCAB = 50368
HIDDEN = 768
EPS = 1e-05

_NC = 2
_NS = 16
_NW = _NC * _NS
_CHUNK = 128


def _sc_gather(table, ids_flat):
    bs = ids_flat.shape[0]
    b_per_w = bs // _NW
    n_chunks = b_per_w // _CHUNK
    mesh = plsc.VectorSubcoreMesh(core_axis_name="c", subcore_axis_name="s")

    @functools.partial(
        pl.kernel,
        mesh=mesh,
        out_type=jax.ShapeDtypeStruct((bs, HIDDEN), jnp.float32),
        scratch_types=[
            pltpu.VMEM((_CHUNK,), jnp.int32),
            pltpu.VMEM((_CHUNK, HIDDEN), jnp.float32),
            pltpu.SemaphoreType.DMA,
        ],
    )
    def gather_kernel(table_hbm, ids_hbm, out_hbm, idx_v, rows_v, sem):
        wid = lax.axis_index("s") * _NC + lax.axis_index("c")
        base = wid * b_per_w

        def body(ci, carry):
            off = base + ci * _CHUNK
            pltpu.sync_copy(ids_hbm.at[pl.ds(off, _CHUNK)], idx_v)
            pltpu.async_copy(table_hbm.at[idx_v], rows_v, sem).wait()
            pltpu.sync_copy(rows_v, out_hbm.at[pl.ds(off, _CHUNK)])
            return carry

        lax.fori_loop(0, n_chunks, body, 0, unroll=False)

    return gather_kernel(table, ids_flat)


def _ln_transpose_body(rows_ref, w_ref, out_ref):
    x = rows_ref[0]
    t = x.shape[0]
    mean = jnp.mean(x, axis=1, keepdims=True)
    zm = x - mean
    var = jnp.mean(zm * zm, axis=1, keepdims=True)
    y = zm * lax.rsqrt(var + EPS)
    eye = jnp.eye(t, dtype=jnp.float32)
    yt = lax.dot_general(y, eye, (((0,), (0,)), ((), ())),
                         preferred_element_type=jnp.float32)
    out_ref[0, :, 0, :] = yt * w_ref[...].reshape(HIDDEN, 1)


def _tc_ln_transpose(rows, ln_weight, b, s):
    t = 256
    grid = (b, s // t)
    rows3 = rows.reshape(b, s, HIDDEN)
    return pl.pallas_call(
        _ln_transpose_body,
        grid=grid,
        in_specs=[
            pl.BlockSpec((1, t, HIDDEN), lambda i, j: (i, j, 0)),
            pl.BlockSpec((1, HIDDEN), lambda i, j: (0, 0)),
        ],
        out_specs=pl.BlockSpec((1, HIDDEN, 1, t), lambda i, j: (i, 0, 0, j)),
        out_shape=jax.ShapeDtypeStruct((b, HIDDEN, 1, s), jnp.float32),
    )(rows3, ln_weight.reshape(1, HIDDEN))


def kernel(input_ids, table, ln_weight):
    b, s = input_ids.shape
    ids_flat = input_ids.reshape(b * s).astype(jnp.int32)
    rows = _sc_gather(table, ids_flat)
    return _tc_ln_transpose(rows, ln_weight, b, s)


# speedup vs baseline: 1.0383x; 1.0383x over previous
import functools

import jax
import jax.numpy as jnp
from jax import lax
from jax.experimental import pallas as pl
from jax.experimental.pallas import tpu as pltpu
from jax.experimental.pallas import tpu_sc as plsc

VOCAB = 50368
HIDDEN = 768
EPS = 1e-05

_NC = 2
_NS = 16
_NW = _NC * _NS
_CHUNK = 128


def _sc_gather(table, ids_flat):
    bs = ids_flat.shape[0]
    b_per_w = bs // _NW
    n_chunks = b_per_w // _CHUNK
    mesh = plsc.VectorSubcoreMesh(core_axis_name="c", subcore_axis_name="s")

    @functools.partial(
        pl.kernel,
        mesh=mesh,
        out_type=jax.ShapeDtypeStruct((bs, HIDDEN), jnp.float32),
        scratch_types=[
            pltpu.VMEM((_CHUNK,), jnp.int32),
            pltpu.VMEM((_CHUNK, HIDDEN), jnp.float32),
            pltpu.SemaphoreType.DMA,
        ],
    )
    def gather_kernel(table_hbm, ids_hbm, out_hbm, idx_v, rows_v, sem):
        wid = lax.axis_index("s") * _NC + lax.axis_index("c")
        base = wid * b_per_w

        def body(ci, carry):
            off = base + ci * _CHUNK
            pltpu.sync_copy(ids_hbm.at[pl.ds(off, _CHUNK)], idx_v)
            pltpu.async_copy(table_hbm.at[idx_v], rows_v, sem).wait()
            pltpu.sync_copy(rows_v, out_hbm.at[pl.ds(off, _CHUNK)])
            return carry

        lax.fori_loop(0, n_chunks, body, 0, unroll=False)

    return gather_kernel(table, ids_flat)


def _ln_transpose_body(rows_ref, w_ref, out_ref):
    x = rows_ref[0]
    t = x.shape[0]
    mean = jnp.mean(x, axis=1, keepdims=True)
    zm = x - mean
    var = jnp.mean(zm * zm, axis=1, keepdims=True)
    y = zm * lax.rsqrt(var + EPS) * w_ref[...]
    out_ref[0, :, 0, :] = y.T


def _tc_ln_transpose(rows, ln_weight, b, s):
    t = 256
    grid = (b, s // t)
    rows3 = rows.reshape(b, s, HIDDEN)
    return pl.pallas_call(
        _ln_transpose_body,
        grid=grid,
        in_specs=[
            pl.BlockSpec((1, t, HIDDEN), lambda i, j: (i, j, 0)),
            pl.BlockSpec((1, HIDDEN), lambda i, j: (0, 0)),
        ],
        out_specs=pl.BlockSpec((1, HIDDEN, 1, t), lambda i, j: (i, 0, 0, j)),
        out_shape=jax.ShapeDtypeStruct((b, HIDDEN, 1, s), jnp.float32),
    )(rows3, ln_weight.reshape(1, HIDDEN))


def kernel(input_ids, table, ln_weight):
    b, s = input_ids.shape
    ids_flat = input_ids.reshape(b * s).astype(jnp.int32)
    rows = _sc_gather(table, ids_flat)
    return _tc_ln_transpose(rows, ln_weight, b, s)


# speedup vs baseline: 1.3879x; 1.3366x over previous
import functools

import jax
import jax.numpy as jnp
from jax import lax
from jax.experimental import pallas as pl
from jax.experimental.pallas import tpu as pltpu
from jax.experimental.pallas import tpu_sc as plsc

VOCAB = 50368
HIDDEN = 768
EPS = 1e-05

_NC = 2
_NS = 16
_NW = _NC * _NS
_CHUNK = 128


def _sc_gather(table, ids_flat):
    bs = ids_flat.shape[0]
    b_per_w = bs // _NW
    n_chunks = b_per_w // _CHUNK
    mesh = plsc.VectorSubcoreMesh(core_axis_name="c", subcore_axis_name="s")

    @functools.partial(
        pl.kernel,
        mesh=mesh,
        out_type=jax.ShapeDtypeStruct((bs, HIDDEN), jnp.float32),
        scratch_types=[
            pltpu.VMEM((_CHUNK,), jnp.int32),
            pltpu.VMEM((_CHUNK, HIDDEN), jnp.float32),
            pltpu.SemaphoreType.DMA,
        ],
    )
    def gather_kernel(table_hbm, ids_hbm, out_hbm, idx_v, rows_v, sem):
        wid = lax.axis_index("s") * _NC + lax.axis_index("c")
        base = wid * b_per_w

        def body(ci, carry):
            off = base + ci * _CHUNK
            pltpu.sync_copy(ids_hbm.at[pl.ds(off, _CHUNK)], idx_v)
            pltpu.async_copy(table_hbm.at[idx_v], rows_v, sem).wait()
            pltpu.sync_copy(rows_v, out_hbm.at[pl.ds(off, _CHUNK)])
            return carry

        lax.fori_loop(0, n_chunks, body, 0, unroll=False)

    return gather_kernel(table, ids_flat)


def _ln_transpose_body(rows_ref, w_ref, out_ref):
    x = rows_ref[0]
    t = x.shape[0]
    mean = jnp.mean(x, axis=1, keepdims=True)
    zm = x - mean
    var = jnp.mean(zm * zm, axis=1, keepdims=True)
    y = zm * lax.rsqrt(var + EPS) * w_ref[...]
    out_ref[0, :, 0, :] = y.T


def _tc_ln_transpose(rows, ln_weight, b, s):
    t = 1024
    grid = (b, s // t)
    rows3 = rows.reshape(b, s, HIDDEN)
    return pl.pallas_call(
        _ln_transpose_body,
        grid=grid,
        in_specs=[
            pl.BlockSpec((1, t, HIDDEN), lambda i, j: (i, j, 0)),
            pl.BlockSpec((1, HIDDEN), lambda i, j: (0, 0)),
        ],
        out_specs=pl.BlockSpec((1, HIDDEN, 1, t), lambda i, j: (i, 0, 0, j)),
        out_shape=jax.ShapeDtypeStruct((b, HIDDEN, 1, s), jnp.float32),
    )(rows3, ln_weight.reshape(1, HIDDEN))


def kernel(input_ids, table, ln_weight):
    b, s = input_ids.shape
    ids_flat = input_ids.reshape(b * s).astype(jnp.int32)
    rows = _sc_gather(table, ids_flat)
    return _tc_ln_transpose(rows, ln_weight, b, s)


# speedup vs baseline: 1.4605x; 1.0523x over previous
import functools

import jax
import jax.numpy as jnp
from jax import lax
from jax.experimental import pallas as pl
from jax.experimental.pallas import tpu as pltpu
from jax.experimental.pallas import tpu_sc as plsc

VOCAB = 50368
HIDDEN = 768
EPS = 1e-05

_NC = 2
_NS = 16
_NW = _NC * _NS
_CHUNK = 128


def _sc_gather(table, ids_flat):
    bs = ids_flat.shape[0]
    b_per_w = bs // _NW
    n_chunks = b_per_w // _CHUNK
    mesh = plsc.VectorSubcoreMesh(core_axis_name="c", subcore_axis_name="s")

    @functools.partial(
        pl.kernel,
        mesh=mesh,
        out_type=jax.ShapeDtypeStruct((bs, HIDDEN), jnp.float32),
        scratch_types=[
            pltpu.VMEM((_CHUNK,), jnp.int32),
            pltpu.VMEM((_CHUNK, HIDDEN), jnp.float32),
            pltpu.SemaphoreType.DMA,
        ],
    )
    def gather_kernel(table_hbm, ids_hbm, out_hbm, idx_v, rows_v, sem):
        wid = lax.axis_index("s") * _NC + lax.axis_index("c")
        base = wid * b_per_w

        def body(ci, carry):
            off = base + ci * _CHUNK
            pltpu.sync_copy(ids_hbm.at[pl.ds(off, _CHUNK)], idx_v)
            pltpu.async_copy(table_hbm.at[idx_v], rows_v, sem).wait()
            pltpu.sync_copy(rows_v, out_hbm.at[pl.ds(off, _CHUNK)])
            return carry

        lax.fori_loop(0, n_chunks, body, 0, unroll=False)

    return gather_kernel(table, ids_flat)


def _ln_transpose_body(rows_ref, w_ref, out_ref):
    x = rows_ref[0]
    t = x.shape[0]
    mean = jnp.mean(x, axis=1, keepdims=True)
    zm = x - mean
    var = jnp.mean(zm * zm, axis=1, keepdims=True)
    y = zm * lax.rsqrt(var + EPS) * w_ref[...]
    out_ref[0, :, 0, :] = y.T


def _tc_ln_transpose(rows, ln_weight, b, s):
    t = 2048
    grid = (b, s // t)
    rows3 = rows.reshape(b, s, HIDDEN)
    return pl.pallas_call(
        _ln_transpose_body,
        grid=grid,
        in_specs=[
            pl.BlockSpec((1, t, HIDDEN), lambda i, j: (i, j, 0)),
            pl.BlockSpec((1, HIDDEN), lambda i, j: (0, 0)),
        ],
        out_specs=pl.BlockSpec((1, HIDDEN, 1, t), lambda i, j: (i, 0, 0, j)),
        out_shape=jax.ShapeDtypeStruct((b, HIDDEN, 1, s), jnp.float32),
    )(rows3, ln_weight.reshape(1, HIDDEN))


def kernel(input_ids, table, ln_weight):
    b, s = input_ids.shape
    ids_flat = input_ids.reshape(b * s).astype(jnp.int32)
    rows = _sc_gather(table, ids_flat)
    return _tc_ln_transpose(rows, ln_weight, b, s)
